# baseline (speedup 1.0000x reference)
import functools

import jax
import jax.numpy as jnp
from jax.experimental import pallas as pl
from jax.experimental.pallas import tpu as pltpu

_EPS = 1e-5


def _fold_w(w, W):
    Cin, Cout = int(w.shape[3]), int(w.shape[4])
    wo = jnp.arange(W)[:, None, None]
    wi = jnp.arange(W)[None, :, None]
    kw = jnp.arange(3)[None, None, :]
    band = (wi == wo + kw - 1).astype(w.dtype)
    t = jnp.einsum('xik,dhkco->dichxo', band, w)
    return t.reshape(3, W * Cin, 3 * W * Cout).astype(jnp.bfloat16)


def _bn_terms(s, q, gamma, beta, W, Cout, count):
    ts = s.sum(axis=(0, 1)).reshape(W, Cout).sum(axis=0)
    tq = q.sum(axis=(0, 1)).reshape(W, Cout).sum(axis=0)
    mean = ts / count
    var = jnp.maximum(tq / count - mean * mean, 0.0)
    scale = gamma / jnp.sqrt(var + _EPS)
    shift = beta - mean * scale
    return scale, shift


def _tile_lanes(v, W):
    return jnp.tile(v.astype(jnp.float32), W).reshape(1, -1)


def _conv_kernel(*refs, n_in, norm, D, H, N):
    xs = list(refs[:n_in])
    ws = refs[n_in:2 * n_in]
    pos = 2 * n_in
    scsh = []
    for j in range(n_in):
        if norm[j]:
            scsh.append((refs[pos], refs[pos + 1]))
            pos += 2
        else:
            scsh.append(None)
    y_ref, s_ref, q_ref = refs[pos:pos + 3]
    scratch = refs[pos + 3:]

    si = 0
    for j in range(n_in):
        if norm[j]:
            sc, sh = scsh[j]
            for d in range(D):
                v = xs[j][d].astype(jnp.float32) * sc[...] + sh[...]
                scratch[si][d] = jnp.maximum(v, 0.0).astype(jnp.bfloat16)
            xs[j] = scratch[si]
            si += 1

    s = jnp.zeros((1, N), jnp.float32)
    q = jnp.zeros((1, N), jnp.float32)
    zrow = jnp.zeros((1, N), jnp.float32)
    for d in range(D):
        t = jnp.zeros((H, 3 * N), jnp.float32)
        for kd in range(3):
            di = d + kd - 1
            if 0 <= di < D:
                for j in range(n_in):
                    t = t + jnp.dot(xs[j][di], ws[j][kd],
                                    preferred_element_type=jnp.float32)
        acc = t[:, N:2 * N]
        acc = acc + jnp.concatenate([zrow, t[:-1, :N]], axis=0)
        acc = acc + jnp.concatenate([t[1:, 2 * N:], zrow], axis=0)
        y_ref[d] = acc.astype(y_ref.dtype)
        s = s + jnp.sum(acc, axis=0, keepdims=True)
        q = q + jnp.sum(acc * acc, axis=0, keepdims=True)
    s_ref[...] = s
    q_ref[...] = q


def _bnpool_kernel(y_ref, sc_ref, sh_ref, o_ref, *, D, H, W, C):
    for do in range(D // 2):
        v0 = y_ref[2 * do].astype(jnp.float32) * sc_ref[...] + sh_ref[...]
        v1 = y_ref[2 * do + 1].astype(jnp.float32) * sc_ref[...] + sh_ref[...]
        m = jnp.maximum(jnp.maximum(v0, v1), 0.0)
        m = m.reshape(H // 2, 2, W * C).max(axis=1)
        m = m.reshape(H // 2, W // 2, 2, C).max(axis=2)
        o_ref[do] = m.reshape(H // 2, (W // 2) * C).astype(o_ref.dtype)


def _up_kernel(a_ref, sc_ref, sh_ref, w_ref, b_ref, o_ref):
    v = jnp.maximum(a_ref[...].astype(jnp.float32) * sc_ref[...] + sh_ref[...],
                    0.0)
    acc = jnp.dot(v.astype(jnp.bfloat16), w_ref[...],
                  preferred_element_type=jnp.float32)
    o_ref[...] = (acc + b_ref[...]).astype(o_ref.dtype)


def _final_kernel(y_ref, sc_ref, sh_ref, fw_ref, fb_ref, o_ref, *, D, H, W, C):
    for d in range(D):
        v = jnp.maximum(y_ref[d].astype(jnp.float32) * sc_ref[...] + sh_ref[...],
                        0.0)
        z = (v * fw_ref[...]).reshape(H, W, C).sum(axis=2) + fb_ref[0, 0]
        o_ref[d] = 1.0 / (1.0 + jnp.exp(-z))


def _conv_pass(xs, ws, norms, gamma, beta, W):
    Nb, D, H, _ = xs[0].shape
    cins = [int(x.shape[-1]) // W for x in xs]
    Cout = int(ws[0].shape[-1])
    N = W * Cout
    n_in = len(xs)
    norm = [n is not None for n in norms]

    wts = [_fold_w(w, W) for w in ws]
    extra = []
    scratch = []
    for j, nrm in enumerate(norms):
        if nrm is not None:
            extra.append(_tile_lanes(nrm[0], W))
            extra.append(_tile_lanes(nrm[1], W))
            scratch.append(pltpu.VMEM((D, H, W * cins[j]), jnp.bfloat16))

    in_specs = (
        [pl.BlockSpec((None, D, H, W * c), lambda n: (n, 0, 0, 0)) for c in cins]
        + [pl.BlockSpec((3, W * c, 3 * N), lambda n: (0, 0, 0)) for c in cins]
    )
    for j, c in enumerate(cins):
        if norm[j]:
            in_specs.append(pl.BlockSpec((1, W * c), lambda n: (0, 0)))
            in_specs.append(pl.BlockSpec((1, W * c), lambda n: (0, 0)))

    y, s, q = pl.pallas_call(
        functools.partial(_conv_kernel, n_in=n_in, norm=norm, D=D, H=H, N=N),
        grid=(Nb,),
        in_specs=in_specs,
        out_specs=(pl.BlockSpec((None, D, H, N), lambda n: (n, 0, 0, 0)),
                   pl.BlockSpec((None, 1, N), lambda n: (n, 0, 0)),
                   pl.BlockSpec((None, 1, N), lambda n: (n, 0, 0))),
        out_shape=(jax.ShapeDtypeStruct((Nb, D, H, N), jnp.bfloat16),
                   jax.ShapeDtypeStruct((Nb, 1, N), jnp.float32),
                   jax.ShapeDtypeStruct((Nb, 1, N), jnp.float32)),
        scratch_shapes=scratch,
        compiler_params=pltpu.CompilerParams(
            dimension_semantics=("parallel",)),
    )(*xs, *wts, *extra)

    scale, shift = _bn_terms(s, q, gamma, beta, W, Cout,
                             float(Nb * D * H * W))
    return y, scale, shift


def _block(xs, norms, w1, g1, b1, w2, g2, b2, W):
    cins = [int(x.shape[-1]) // W for x in xs]
    if len(cins) > 1:
        off = 0
        w1s = []
        for c in cins:
            w1s.append(w1[:, :, :, off:off + c, :])
            off += c
    else:
        w1s = [w1]
    y1, sc1, sh1 = _conv_pass(xs, w1s, norms, g1, b1, W)
    return _conv_pass([y1], [w2], [(sc1, sh1)], g2, b2, W)


def _bnpool(y, scale, shift, W, C):
    Nb, D, H, _ = y.shape
    return pl.pallas_call(
        functools.partial(_bnpool_kernel, D=D, H=H, W=W, C=C),
        grid=(Nb,),
        in_specs=[pl.BlockSpec((None, D, H, W * C), lambda n: (n, 0, 0, 0)),
                  pl.BlockSpec((1, W * C), lambda n: (0, 0)),
                  pl.BlockSpec((1, W * C), lambda n: (0, 0))],
        out_specs=pl.BlockSpec((None, D // 2, H // 2, (W // 2) * C),
                               lambda n: (n, 0, 0, 0)),
        out_shape=jax.ShapeDtypeStruct((Nb, D // 2, H // 2, (W // 2) * C),
                                       jnp.bfloat16),
        compiler_params=pltpu.CompilerParams(
            dimension_semantics=("parallel",)),
    )(y, _tile_lanes(scale, W), _tile_lanes(shift, W))


def _upsample(y, scale, shift, w, b, W, C):
    Nb, D, H, _ = y.shape
    Cout = int(w.shape[-1])
    M = Nb * D * H * W
    tm = 512 if M % 512 == 0 else M
    a = y.reshape(M, C)
    wm = w.reshape(C, 8 * Cout).astype(jnp.bfloat16)
    bias = jnp.tile(b.astype(jnp.float32), 8).reshape(1, 8 * Cout)
    z = pl.pallas_call(
        _up_kernel,
        grid=(M // tm,),
        in_specs=[pl.BlockSpec((tm, C), lambda i: (i, 0)),
                  pl.BlockSpec((1, C), lambda i: (0, 0)),
                  pl.BlockSpec((1, C), lambda i: (0, 0)),
                  pl.BlockSpec((C, 8 * Cout), lambda i: (0, 0)),
                  pl.BlockSpec((1, 8 * Cout), lambda i: (0, 0))],
        out_specs=pl.BlockSpec((tm, 8 * Cout), lambda i: (i, 0)),
        out_shape=jax.ShapeDtypeStruct((M, 8 * Cout), jnp.bfloat16),
        compiler_params=pltpu.CompilerParams(
            dimension_semantics=("parallel",)),
    )(a, scale.astype(jnp.float32).reshape(1, C),
      shift.astype(jnp.float32).reshape(1, C), wm, bias)
    z = z.reshape(Nb, D, H, W, 2, 2, 2, Cout)
    z = z.transpose(0, 1, 4, 2, 5, 3, 6, 7)
    return z.reshape(Nb, 2 * D, 2 * H, (2 * W) * Cout)


def _final(y, scale, shift, fw, fb, W, C):
    Nb, D, H, _ = y.shape
    fwt = _tile_lanes(fw[:, 0], W)
    out = pl.pallas_call(
        functools.partial(_final_kernel, D=D, H=H, W=W, C=C),
        grid=(Nb,),
        in_specs=[pl.BlockSpec((None, D, H, W * C), lambda n: (n, 0, 0, 0)),
                  pl.BlockSpec((1, W * C), lambda n: (0, 0)),
                  pl.BlockSpec((1, W * C), lambda n: (0, 0)),
                  pl.BlockSpec((1, W * C), lambda n: (0, 0)),
                  pl.BlockSpec((1, 1), lambda n: (0, 0))],
        out_specs=pl.BlockSpec((None, D, H, W), lambda n: (n, 0, 0, 0)),
        out_shape=jax.ShapeDtypeStruct((Nb, D, H, W), jnp.float32),
        compiler_params=pltpu.CompilerParams(
            dimension_semantics=("parallel",)),
    )(y, _tile_lanes(scale, W), _tile_lanes(shift, W), fwt,
      fb.astype(jnp.float32).reshape(1, 1))
    return out.reshape(Nb, 1, D, H, W)


def kernel(enc1_w1, enc1_g1, enc1_b1, enc1_w2, enc1_g2, enc1_b2,
           enc2_w1, enc2_g1, enc2_b1, enc2_w2, enc2_g2, enc2_b2,
           enc3_w1, enc3_g1, enc3_b1, enc3_w2, enc3_g2, enc3_b2,
           enc4_w1, enc4_g1, enc4_b1, enc4_w2, enc4_g2, enc4_b2,
           bottleneck_w1, bottleneck_g1, bottleneck_b1,
           bottleneck_w2, bottleneck_g2, bottleneck_b2,
           up4_w, up4_b,
           dec4_w1, dec4_g1, dec4_b1, dec4_w2, dec4_g2, dec4_b2,
           up3_w, up3_b,
           dec3_w1, dec3_g1, dec3_b1, dec3_w2, dec3_g2, dec3_b2,
           up2_w, up2_b,
           dec2_w1, dec2_g1, dec2_b1, dec2_w2, dec2_g2, dec2_b2,
           up1_w, up1_b,
           dec1_w1, dec1_g1, dec1_b1, dec1_w2, dec1_g2, dec1_b2,
           final_w, final_b, x):
    Nb = x.shape[0]
    xp = jnp.transpose(x, (0, 2, 3, 4, 1)).reshape(
        Nb, 64, 64, 64 * 2).astype(jnp.bfloat16)

    ye1, se1, he1 = _block([xp], [None], enc1_w1, enc1_g1, enc1_b1,
                           enc1_w2, enc1_g2, enc1_b2, 64)
    p1 = _bnpool(ye1, se1, he1, 64, 8)
    ye2, se2, he2 = _block([p1], [None], enc2_w1, enc2_g1, enc2_b1,
                           enc2_w2, enc2_g2, enc2_b2, 32)
    p2 = _bnpool(ye2, se2, he2, 32, 16)
    ye3, se3, he3 = _block([p2], [None], enc3_w1, enc3_g1, enc3_b1,
                           enc3_w2, enc3_g2, enc3_b2, 16)
    p3 = _bnpool(ye3, se3, he3, 16, 32)
    ye4, se4, he4 = _block([p3], [None], enc4_w1, enc4_g1, enc4_b1,
                           enc4_w2, enc4_g2, enc4_b2, 8)
    p4 = _bnpool(ye4, se4, he4, 8, 64)
    yb, sb, hb = _block([p4], [None], bottleneck_w1, bottleneck_g1,
                        bottleneck_b1, bottleneck_w2, bottleneck_g2,
                        bottleneck_b2, 4)

    u4 = _upsample(yb, sb, hb, up4_w, up4_b, 4, 128)
    yd4, sd4, hd4 = _block([u4, ye4], [None, (se4, he4)],
                           dec4_w1, dec4_g1, dec4_b1,
                           dec4_w2, dec4_g2, dec4_b2, 8)
    u3 = _upsample(yd4, sd4, hd4, up3_w, up3_b, 8, 64)
    yd3, sd3, hd3 = _block([u3, ye3], [None, (se3, he3)],
                           dec3_w1, dec3_g1, dec3_b1,
                           dec3_w2, dec3_g2, dec3_b2, 16)
    u2 = _upsample(yd3, sd3, hd3, up2_w, up2_b, 16, 32)
    yd2, sd2, hd2 = _block([u2, ye2], [None, (se2, he2)],
                           dec2_w1, dec2_g1, dec2_b1,
                           dec2_w2, dec2_g2, dec2_b2, 32)
    u1 = _upsample(yd2, sd2, hd2, up1_w, up1_b, 32, 16)
    yd1, sd1, hd1 = _block([u1, ye1], [None, (se1, he1)],
                           dec1_w1, dec1_g1, dec1_b1,
                           dec1_w2, dec1_g2, dec1_b2, 64)

    return _final(yd1, sd1, hd1, final_w, final_b, 64, 8)
